# baseline (speedup 1.0000x reference)
import jax
import jax.numpy as jnp
from jax import lax
from jax.experimental import pallas as pl
from jax.experimental.pallas import tpu as pltpu
from jax.experimental.pallas import tpu_sc as plsc

N = 50000
E = 800000
H = 64
OUT = 2

NC = 2
NS = 16
NW = NC * NS

NPAD = 50048
RPT = NPAD // NW
DPAD = NPAD - 1

CH = 512
IB = CH // 128
EP = 802816
SCH = EP // NW // CH
FB = 2048
CAP = EP + FB
CK = 256

_mesh = plsc.VectorSubcoreMesh(
    core_axis_name="c", subcore_axis_name="s", num_cores=NC, num_subcores=NS)
_sc_params = pltpu.CompilerParams(use_tc_tiling_on_sc=False)
_sc_params_sort = pltpu.CompilerParams(
    use_tc_tiling_on_sc=False, needs_layout_passes=False)


def _wid():
    return lax.axis_index("s") * NC + lax.axis_index("c")


def _kb_body(dst_hbm, ids_hbm, cnt_hbm, dst_v, idb, cnt_v, off_s):
    b = _wid()
    off_s[0] = 0
    off_s[1] = 0
    iota16 = lax.iota(jnp.int32, 16)

    def chunk(i, _):
        pltpu.sync_copy(dst_hbm.at[pl.ds(i * IB, IB)], dst_v)
        for q in range(IB):
            for t in range(8):
                d16 = dst_v[q, pl.ds(t * 16, 16)]
                matches = (d16 & 31) == b
                key16 = jnp.where(matches, 0, 1).astype(jnp.int32)
                eid16 = iota16 + (i * CH + q * 128 + t * 16)
                packed = (lax.shift_right_logical(d16, 5) << 20) | eid16
                svals = plsc.sort_key_val(key16, packed)[1]
                npop = 16 - jnp.sum(key16, axis=0)
                off = off_s[0]
                idb[pl.ds(off, 16)] = svals
                off_s[0] = off + npop

                @pl.when(off + npop >= FB)
                def _flush():
                    fo = off_s[1]
                    pltpu.sync_copy(idb.at[pl.ds(0, FB)],
                                    ids_hbm.at[b, pl.ds(fo * FB, FB)])
                    resid_i = idb[pl.ds(FB, 16)]
                    idb[pl.ds(0, 16)] = resid_i
                    off_s[1] = fo + 1
                    off_s[0] = off + npop - FB
        return _

    lax.fori_loop(0, EP // CH, chunk, None)
    fo = off_s[1]
    pltpu.sync_copy(idb.at[pl.ds(0, FB)], ids_hbm.at[b, pl.ds(fo * FB, FB)])
    total = fo * FB + off_s[0]
    for t in range(8):
        cnt_v[pl.ds(t * 16, 16)] = jnp.zeros((16,), jnp.int32) + total
    pltpu.sync_copy(cnt_v, cnt_hbm.at[b])


def _kb(dst2d):
    return pl.kernel(
        _kb_body,
        out_type=(jax.ShapeDtypeStruct((NW, CAP), jnp.int32),
                  jax.ShapeDtypeStruct((NW, 128), jnp.int32)),
        mesh=_mesh,
        scratch_types=[
            pltpu.VMEM((IB, 128), jnp.int32),
            pltpu.VMEM((FB + 16, ), jnp.int32),
            pltpu.VMEM((128,), jnp.int32),
            pltpu.SMEM((2,), jnp.int32),
        ],
        compiler_params=_sc_params_sort,
    )(dst2d)


def _prep_body(x_ref, p_ref, w0h_ref, w0p_ref, b0_ref, w1p_ref, w2p_ref,
               g0_ref, n0_ref, n1_ref, n2_ref, q1_ref, q2_ref):
    x = x_ref[...]
    p = p_ref[...]
    p0 = p @ w0p_ref[...]
    p1 = p @ w1p_ref[...]
    p2 = p @ w2p_ref[...]
    g0_ref[...] = x @ w0h_ref[...] + p0 + b0_ref[...]
    n0_ref[...] = -p0
    n1_ref[...] = -p1
    n2_ref[...] = -p2
    q1_ref[...] = p1
    q2_ref[...] = p2


def _prep(xp, posp, W0a, b0a, W1a, W2a):
    BRP = 3128
    out = jax.ShapeDtypeStruct((NPAD, H), jnp.float32)
    return pl.pallas_call(
        _prep_body,
        grid=(NPAD // BRP,),
        in_specs=[
            pl.BlockSpec((BRP, 8), lambda i: (i, 0)),
            pl.BlockSpec((BRP, 3), lambda i: (i, 0)),
            pl.BlockSpec((8, H), lambda i: (0, 0)),
            pl.BlockSpec((3, H), lambda i: (0, 0)),
            pl.BlockSpec((H,), lambda i: (0,)),
            pl.BlockSpec((3, H), lambda i: (0, 0)),
            pl.BlockSpec((3, H), lambda i: (0, 0)),
        ],
        out_specs=[pl.BlockSpec((BRP, H), lambda i: (i, 0))] * 6,
        out_shape=[out] * 6,
    )(xp, posp, W0a[:8], W0a[8:], b0a, W1a[64:], W2a[64:])


def _s1_body(g_hbm, np_hbm, src_hbm, dst_hbm, out_hbm,
             src_v, dst_v, g_v, p_v, sem):
    rbase = _wid() * (SCH * IB)

    def chunk(i, _):
        roff = rbase + i * IB
        pltpu.sync_copy(src_hbm.at[pl.ds(roff, IB)], src_v)
        pltpu.sync_copy(dst_hbm.at[pl.ds(roff, IB)], dst_v)
        descs = []
        for q in range(IB):
            descs.append(pltpu.async_copy(
                g_hbm.at[src_v.at[q]], g_v.at[pl.ds(q * 128, 128)], sem))
            descs.append(pltpu.async_copy(
                np_hbm.at[dst_v.at[q]], p_v.at[pl.ds(q * 128, 128)], sem))
        for d in descs:
            d.wait()

        def add_row(j, _):
            for c in range(H // 16):
                sl = pl.ds(c * 16, 16)
                g_v[j, sl] = g_v[j, sl] + p_v[j, sl]
            return _
        lax.fori_loop(0, CH, add_row, None)
        pltpu.sync_copy(g_v, out_hbm.at[pl.ds(roff * 128, CH)])
        return _
    lax.fori_loop(0, SCH, chunk, None)


def _s1(G, negP, src2d, dst2d):
    return pl.kernel(
        _s1_body,
        out_type=jax.ShapeDtypeStruct((EP, H), jnp.float32),
        mesh=_mesh,
        scratch_types=[
            pltpu.VMEM((IB, 128), jnp.int32),
            pltpu.VMEM((IB, 128), jnp.int32),
            pltpu.VMEM((CH, H), jnp.float32),
            pltpu.VMEM((CH, H), jnp.float32),
            pltpu.SemaphoreType.DMA,
        ],
        compiler_params=_sc_params,
    )(G, negP, src2d, dst2d)


def _t2_body(pre_ref, wb_ref, bb_ref, o_ref):
    o_ref[...] = jnp.maximum(pre_ref[...], 0.0) @ wb_ref[...] + bb_ref[...]


def _t2(pre, Wb, bb):
    BE = 2048
    return pl.pallas_call(
        _t2_body,
        grid=(EP // BE,),
        in_specs=[
            pl.BlockSpec((BE, H), lambda i: (i, 0)),
            pl.BlockSpec((H, H), lambda i: (0, 0)),
            pl.BlockSpec((H,), lambda i: (0,)),
        ],
        out_specs=pl.BlockSpec((BE, H), lambda i: (i, 0)),
        out_shape=jax.ShapeDtypeStruct((EP, H), jnp.float32),
    )(pre, Wb, bb)


def _s3_body(m_hbm, ids_hbm, cnt_hbm, h_hbm,
             acc, mb, id_v, gx_v, cnt_v, idx_v, sem):
    b = _wid()
    pltpu.sync_copy(cnt_hbm.at[b], cnt_v)
    cnt = cnt_v[pl.ds(0, 16)][0]
    zero16 = jnp.zeros((16,), jnp.float32)
    iota16 = lax.iota(jnp.int32, 16)
    sentinel = ((RPT - 1) << 20) | E

    def zrow(j, _):
        for c in range(H // 16):
            acc[j, pl.ds(c * 16, 16)] = zero16
        return _
    lax.fori_loop(0, RPT, zrow, None)

    nch = (cnt + (CK - 1)) >> 8

    def chunk(ci, _):
        off = ci * CK
        pltpu.sync_copy(ids_hbm.at[b, pl.ds(ci * (CK // 128), CK // 128)], id_v)
        nv = cnt - off
        for q in range(CK // 128):
            for t in range(8):
                sl = pl.ds(t * 16, 16)
                lane = iota16 + (q * 128 + t * 16)
                p16 = jnp.where(lane < nv, id_v[q, sl], sentinel)
                id_v[q, sl] = p16
                gx_v[q, sl] = p16 & 0xFFFFF
        descs = []
        for q in range(CK // 128):
            descs.append(pltpu.async_copy(
                m_hbm.at[gx_v.at[q]], mb.at[pl.ds(q * 128, 128)], sem))
        for d in descs:
            d.wait()

        def grp(g, _):
            r16 = lax.shift_right_logical(
                id_v[g >> 3, pl.ds((g & 7) * 16, 16)], 20)
            for k in range(16):
                r = r16[k]
                j = g * 16 + k
                for c in range(H // 16):
                    sl = pl.ds(c * 16, 16)
                    acc[r, sl] = jnp.maximum(acc[r, sl], mb[j, sl])
            return _
        lax.fori_loop(0, CK // 16, grp, None)
        return _
    lax.fori_loop(0, nch, chunk, None)

    for q in range(13):
        row0 = q * 128 if q < 12 else RPT - 128
        for t in range(8):
            idx_v[q, pl.ds(t * 16, 16)] = (iota16 + (row0 + t * 16)) * 32 + b
    descs = []
    for q in range(13):
        row0 = q * 128 if q < 12 else RPT - 128
        descs.append(pltpu.async_copy(
            acc.at[pl.ds(row0, 128)], h_hbm.at[idx_v.at[q]], sem))
    for d in descs:
        d.wait()


def _s3(m, ids3d, cnt):
    return pl.kernel(
        _s3_body,
        out_type=jax.ShapeDtypeStruct((NPAD, H), jnp.float32),
        mesh=_mesh,
        scratch_types=[
            pltpu.VMEM((RPT, H), jnp.float32),
            pltpu.VMEM((CK, H), jnp.float32),
            pltpu.VMEM((CK // 128, 128), jnp.int32),
            pltpu.VMEM((CK // 128, 128), jnp.int32),
            pltpu.VMEM((128,), jnp.int32),
            pltpu.VMEM((13, 128), jnp.int32),
            pltpu.SemaphoreType.DMA,
        ],
        compiler_params=_sc_params,
    )(m, ids3d, cnt)


def _tcg_body(h_ref, w_ref, q_ref, ba_ref, o_ref):
    o_ref[...] = h_ref[...] @ w_ref[...] + q_ref[...] + ba_ref[...]


def _tcg(h, Wh64, Q, ba):
    BRP = 3128
    return pl.pallas_call(
        _tcg_body,
        grid=(NPAD // BRP,),
        in_specs=[
            pl.BlockSpec((BRP, H), lambda i: (i, 0)),
            pl.BlockSpec((H, H), lambda i: (0, 0)),
            pl.BlockSpec((BRP, H), lambda i: (i, 0)),
            pl.BlockSpec((H,), lambda i: (0,)),
        ],
        out_specs=pl.BlockSpec((BRP, H), lambda i: (i, 0)),
        out_shape=jax.ShapeDtypeStruct((NPAD, H), jnp.float32),
    )(h, Wh64, Q, ba)


def _head_body(h_ref, wh_ref, bh_ref, o_ref):
    o_ref[...] = h_ref[...] @ wh_ref[...] + bh_ref[...]


def _head(h, Wh, bh):
    BR = 2000
    return pl.pallas_call(
        _head_body,
        grid=(N // BR,),
        in_specs=[
            pl.BlockSpec((BR, H), lambda i: (i, 0)),
            pl.BlockSpec((H, OUT), lambda i: (0, 0)),
            pl.BlockSpec((OUT,), lambda i: (0,)),
        ],
        out_specs=pl.BlockSpec((BR, OUT), lambda i: (i, 0)),
        out_shape=jax.ShapeDtypeStruct((N, OUT), jnp.float32),
    )(h, Wh, bh)


def kernel(x, pos, edge_index, W0a, b0a, W0b, b0b, W1a, b1a, W1b, b1b,
           W2a, b2a, W2b, b2b, Wh, bh):
    src = edge_index[0]
    dst = edge_index[1]
    src2d = jnp.pad(src, (0, EP - E)).reshape(EP // 128, 128)
    dst2d = jnp.pad(dst, (0, EP - E),
                    constant_values=DPAD).reshape(EP // 128, 128)
    xp = jnp.pad(x, ((0, NPAD - N), (0, 0)))
    posp = jnp.pad(pos, ((0, NPAD - N), (0, 0)))

    ids, cnt = _kb(dst2d)
    ids3d = ids.reshape(NW, CAP // 128, 128)

    G0, nP0, nP1, nP2, Q1, Q2 = _prep(xp, posp, W0a, b0a, W1a, W2a)

    pre = _s1(G0, nP0, src2d, dst2d)
    m = _t2(pre, W0b, b0b)
    h = _s3(m, ids3d, cnt)

    G1 = _tcg(h, W1a[:64], Q1, b1a)
    pre = _s1(G1, nP1, src2d, dst2d)
    m = _t2(pre, W1b, b1b)
    h = _s3(m, ids3d, cnt)

    G2 = _tcg(h, W2a[:64], Q2, b2a)
    pre = _s1(G2, nP2, src2d, dst2d)
    m = _t2(pre, W2b, b2b)
    h = _s3(m, ids3d, cnt)

    return _head(h, Wh, bh)
